# baseline (speedup 1.0000x reference)
import jax
import jax.numpy as jnp
from jax.experimental import pallas as pl
from jax.experimental.pallas import tpu as pltpu

B, F, D, NC = 128, 768, 4096, 8192
BLK = 512
NBUF = 4
NBLK = NC // BLK


def _centroid_copy(c_hbm, bufs, sems, block, slot):
    return pltpu.make_async_copy(
        c_hbm.at[pl.ds(block * BLK, BLK), :], bufs.at[slot], sems.at[slot])


def _body(x_ref, p_ref, c_hbm, o_ref, h_ref, bufs, sems):
    for s in range(NBUF):
        _centroid_copy(c_hbm, bufs, sems, s, s).start()

    acc = jax.lax.dot_general(
        x_ref[...], p_ref[...], (((1,), (1,)), ((), ())),
        preferred_element_type=jnp.float32)
    h_ref[...] = jnp.sign(acc)

    def step(k, _):
        slot = jax.lax.rem(k, NBUF)
        _centroid_copy(c_hbm, bufs, sems, k, slot).wait()
        o_ref[:, pl.ds(k * BLK, BLK)] = jax.lax.dot_general(
            h_ref[...], bufs[slot], (((1,), (1,)), ((), ())),
            preferred_element_type=jnp.float32)
        nxt = k + NBUF

        @pl.when(nxt < NBLK)
        def _prefetch():
            _centroid_copy(c_hbm, bufs, sems, nxt, slot).start()

        return 0

    jax.lax.fori_loop(0, NBLK, step, 0)


def kernel(x, projection, centroids):
    return pl.pallas_call(
        _body,
        in_specs=[
            pl.BlockSpec(memory_space=pltpu.MemorySpace.VMEM),
            pl.BlockSpec(memory_space=pltpu.MemorySpace.VMEM),
            pl.BlockSpec(memory_space=pltpu.MemorySpace.HBM),
        ],
        out_specs=pl.BlockSpec(memory_space=pltpu.MemorySpace.VMEM),
        out_shape=jax.ShapeDtypeStruct((B, NC), jnp.float32),
        scratch_shapes=[
            pltpu.VMEM((B, D), jnp.float32),
            pltpu.VMEM((NBUF, BLK, D), jnp.float32),
            pltpu.SemaphoreType.DMA((NBUF,)),
        ],
    )(x, projection, centroids)
